# baseline (speedup 1.0000x reference)
import jax
import jax.numpy as jnp
from jax import lax
from jax.experimental import pallas as pl
from jax.experimental.pallas import tpu as pltpu
from jax.experimental.pallas import tpu_sc as plsc

N = 10000
NPAD = 10112
B = 16
E = 320000
D = 128
DRAW = 5000
DRAW_PAD = 5120
NCLS = 4

NC, NS = 2, 16
NW = NC * NS
CHUNK = 128
EDGES_PER_TILE = 10240
NCHUNK = EDGES_PER_TILE // CHUNK
NPAIR = NCHUNK // 2
E_PAD = EDGES_PER_TILE * NW
ROWS_PER_TILE = NPAD // NS
DUMP_ROW = N

import functools


@functools.cache
def _get_mesh():
  return plsc.VectorSubcoreMesh(core_axis_name="c", subcore_axis_name="s",
                                num_cores=NC, num_subcores=NS)


def _sc_degrees(dst_g, dst_x, ones_hbm, zeros_hbm):
  def body(dstg_hbm, dstx_hbm, ones_h, zeros_h, outg, outx,
           acc, dst_v, ones_v):
    cid = lax.axis_index("c")
    sid = lax.axis_index("s")
    tile = cid * NS + sid
    sl = pl.ds(sid * ROWS_PER_TILE, ROWS_PER_TILE)
    pltpu.sync_copy(ones_h, ones_v)

    def one_set(dst_hbm, out):
      pltpu.sync_copy(zeros_h, acc.at[sl])
      plsc.subcore_barrier()

      def loop(i, carry):
        base = pl.multiple_of(tile * EDGES_PER_TILE + i * CHUNK, CHUNK)
        pltpu.sync_copy(dst_hbm.at[pl.ds(base, CHUNK)], dst_v)
        pltpu.sync_copy(ones_v, acc.at[dst_v], add=True)
        return carry

      lax.fori_loop(0, NCHUNK, loop, 0)
      plsc.subcore_barrier()
      pltpu.sync_copy(acc.at[sl], out.at[cid, sl])

    one_set(dstg_hbm, outg)
    one_set(dstx_hbm, outx)

  f = pl.kernel(
      body,
      out_type=(jax.ShapeDtypeStruct((NC, NPAD, D), jnp.float32),
                jax.ShapeDtypeStruct((NC, NPAD, D), jnp.float32)),
      mesh=_get_mesh(),
      scratch_types=[
          pltpu.VMEM_SHARED((NPAD, D), jnp.float32),
          pltpu.VMEM((CHUNK,), jnp.int32),
          pltpu.VMEM((CHUNK, D), jnp.float32),
      ],
  )
  return f(dst_g, dst_x, ones_hbm, zeros_hbm)


def _sc_aggregate(y, src, dst, zeros_hbm):
  def body(y_hbm, src_hbm, dst_hbm, zeros_h, out, acc,
           s0, d0, r0, s1, d1, r1, sem0, sem1):
    cid = lax.axis_index("c")
    sid = lax.axis_index("s")
    tile = cid * NS + sid
    sl = pl.ds(sid * ROWS_PER_TILE, ROWS_PER_TILE)
    pltpu.sync_copy(zeros_h, acc.at[sl])
    plsc.subcore_barrier()
    base0 = tile * EDGES_PER_TILE

    def fetch(c, s_v, d_v, r_v, sem):
      off = pl.multiple_of(base0 + c * CHUNK, CHUNK)
      pltpu.sync_copy(src_hbm.at[pl.ds(off, CHUNK)], s_v)
      pltpu.sync_copy(dst_hbm.at[pl.ds(off, CHUNK)], d_v)
      pltpu.async_copy(y_hbm.at[s_v], r_v, sem)

    fetch(0, s0, d0, r0, sem0)

    def loop(g, carry):
      fetch(2 * g + 1, s1, d1, r1, sem1)
      pltpu.make_async_copy(y_hbm.at[s0], r0, sem0).wait()
      pltpu.sync_copy(r0, acc.at[d0], add=True)

      @pl.when(g < NPAIR - 1)
      def _():
        fetch(2 * g + 2, s0, d0, r0, sem0)

      pltpu.make_async_copy(y_hbm.at[s1], r1, sem1).wait()
      pltpu.sync_copy(r1, acc.at[d1], add=True)
      return carry

    lax.fori_loop(0, NPAIR, loop, 0)
    plsc.subcore_barrier()
    pltpu.sync_copy(acc.at[sl], out.at[cid, sl])

  f = pl.kernel(
      body,
      out_type=jax.ShapeDtypeStruct((NC, NPAD, D), jnp.float32),
      mesh=_get_mesh(),
      scratch_types=[
          pltpu.VMEM_SHARED((NPAD, D), jnp.float32),
          pltpu.VMEM((CHUNK,), jnp.int32),
          pltpu.VMEM((CHUNK,), jnp.int32),
          pltpu.VMEM((CHUNK, D), jnp.float32),
          pltpu.VMEM((CHUNK,), jnp.int32),
          pltpu.VMEM((CHUNK,), jnp.int32),
          pltpu.VMEM((CHUNK, D), jnp.float32),
          pltpu.SemaphoreType.DMA,
          pltpu.SemaphoreType.DMA,
      ],
  )
  return f(y, src, dst, zeros_hbm)


def _mm(a, b, *, bm, bn, bk, bias=None, row_scale=None, act=None,
        alpha=None, n_valid=None):
  M, Ka = a.shape
  Kb, Nn = b.shape
  K = max(Ka, Kb)
  nk = -(-K // bk)
  assert M % bm == 0
  grid = (M // bm, -(-Nn // bn), nk)

  def kern(*refs):
    k = pl.program_id(2)
    it = iter(refs)
    alpha_ref = next(it) if act == "prelu" else None
    a_ref = next(it)
    b_ref = next(it)
    rs_ref = next(it) if row_scale is not None else None
    bias_ref = next(it) if bias is not None else None
    out_ref = next(it)

    @pl.when(k == 0)
    def _():
      out_ref[...] = jnp.zeros_like(out_ref)

    ab = a_ref[...]
    bb = b_ref[...]
    if Ka < nk * bk:
      rem = Ka - (nk - 1) * bk
      lim = jnp.where(k == nk - 1, rem, bk)
      col = lax.broadcasted_iota(jnp.int32, ab.shape, 1)
      ab = jnp.where(col < lim, ab, 0.0)
    if Kb < nk * bk:
      rem = Kb - (nk - 1) * bk
      lim = jnp.where(k == nk - 1, rem, bk)
      row = lax.broadcasted_iota(jnp.int32, bb.shape, 0)
      bb = jnp.where(row < lim, bb, 0.0)

    out_ref[...] += jnp.dot(ab, bb, preferred_element_type=jnp.float32)

    @pl.when(k == nk - 1)
    def _():
      acc = out_ref[...]
      if rs_ref is not None:
        acc = acc * rs_ref[...][:, :1]
      if bias_ref is not None:
        acc = acc + bias_ref[...]
      if act == "prelu":
        al = alpha_ref[0, 0]
        acc = jnp.where(acc >= 0, acc, al * acc)
      elif act == "log_softmax":
        colv = lax.broadcasted_iota(jnp.int32, acc.shape, 1)
        valid = colv < n_valid
        z = jnp.where(valid, acc, -jnp.inf)
        m = jnp.max(z, axis=1, keepdims=True)
        e = jnp.where(valid, jnp.exp(z - m), 0.0)
        lse = m + jnp.log(jnp.sum(e, axis=1, keepdims=True))
        acc = z - lse
      out_ref[...] = acc

  in_specs = []
  ops = []
  if act == "prelu":
    in_specs.append(pl.BlockSpec(memory_space=pltpu.SMEM))
    ops.append(alpha)
  in_specs += [
      pl.BlockSpec((bm, bk), lambda i, j, k: (i, k)),
      pl.BlockSpec((bk, bn), lambda i, j, k: (k, j)),
  ]
  ops += [a, b]
  if row_scale is not None:
    in_specs.append(pl.BlockSpec((bm, 128), lambda i, j, k: (i, 0)))
    ops.append(row_scale)
  if bias is not None:
    in_specs.append(pl.BlockSpec((1, bn), lambda i, j, k: (0, j)))
    ops.append(bias)

  return pl.pallas_call(
      kern,
      grid=grid,
      in_specs=in_specs,
      out_specs=pl.BlockSpec((bm, bn), lambda i, j, k: (i, j)),
      out_shape=jax.ShapeDtypeStruct((M, Nn), jnp.float32),
      compiler_params=pltpu.CompilerParams(
          dimension_semantics=("parallel", "parallel", "arbitrary")),
  )(*ops)


def _onehot_and_invcnt(batch_pad):
  ncol = NPAD // 128

  def kern(ids_ref, oh_ref, cnt_ref):
    j = pl.program_id(0)
    ids = ids_ref[...]
    row = lax.broadcasted_iota(jnp.int32, (B, 128), 0)
    oh = (ids == row).astype(jnp.float32)
    oh_ref[...] = oh

    @pl.when(j == 0)
    def _():
      cnt_ref[...] = jnp.zeros_like(cnt_ref)

    cnt_ref[...] += jnp.sum(oh, axis=1, keepdims=True)

    @pl.when(j == ncol - 1)
    def _():
      cnt_ref[...] = 1.0 / jnp.maximum(cnt_ref[...], 1.0)

  return pl.pallas_call(
      kern,
      grid=(ncol,),
      in_specs=[pl.BlockSpec((1, 128), lambda j: (0, j))],
      out_specs=[pl.BlockSpec((B, 128), lambda j: (0, j)),
                 pl.BlockSpec((B, 128), lambda j: (0, 0))],
      out_shape=[jax.ShapeDtypeStruct((B, NPAD), jnp.float32),
                 jax.ShapeDtypeStruct((B, 128), jnp.float32)],
      compiler_params=pltpu.CompilerParams(
          dimension_semantics=("arbitrary",)),
  )(batch_pad)


def _rootsel(rootindex_2d):
  ncol = NPAD // 128

  def kern(root_ref, sel_ref):
    j = pl.program_id(0)
    roots = jnp.stack([root_ref[0, b] for b in range(B)])
    col = lax.broadcasted_iota(jnp.int32, (B, 128), 1) + j * 128
    sel_ref[...] = (col == roots[:, None]).astype(jnp.float32)

  return pl.pallas_call(
      kern,
      grid=(ncol,),
      in_specs=[pl.BlockSpec(memory_space=pltpu.SMEM)],
      out_specs=pl.BlockSpec((B, 128), lambda j: (0, j)),
      out_shape=jax.ShapeDtypeStruct((B, NPAD), jnp.float32),
  )(rootindex_2d)


def _dinv_bcast(p0, p1):
  bm = 1264

  def kern(a_ref, b_ref, o_ref):
    o_ref[...] = lax.rsqrt(a_ref[...] + b_ref[...] + 1.0)

  return pl.pallas_call(
      kern,
      grid=(NPAD // bm,),
      in_specs=[pl.BlockSpec((bm, D), lambda i: (i, 0))] * 2,
      out_specs=pl.BlockSpec((bm, D), lambda i: (i, 0)),
      out_shape=jax.ShapeDtypeStruct((NPAD, D), jnp.float32),
  )(p0, p1)


def _conv_epilogue(p0, p1, y, dinv_b, bias):
  bm = 1264

  def kern(p0_ref, p1_ref, y_ref, d_ref, b_ref, o_ref):
    i = pl.program_id(0)
    s = (p0_ref[...] + p1_ref[...] + y_ref[...]) * d_ref[...]
    v = jnp.maximum(s + b_ref[...], 0.0)
    rowg = lax.broadcasted_iota(jnp.int32, v.shape, 0) + i * bm
    o_ref[...] = jnp.where(rowg < N, v, 0.0)

  return pl.pallas_call(
      kern,
      grid=(NPAD // bm,),
      in_specs=[pl.BlockSpec((bm, D), lambda i: (i, 0))] * 4 +
               [pl.BlockSpec((1, D), lambda i: (0, 0))],
      out_specs=pl.BlockSpec((bm, D), lambda i: (i, 0)),
      out_shape=jax.ShapeDtypeStruct((NPAD, D), jnp.float32),
  )(p0, p1, y, dinv_b, bias)


def _segment_max(h, oh):
  def kern(h_ref, oh_ref, o_ref):
    hb = h_ref[...]
    rows = []
    for b in range(B):
      mask = oh_ref[b, :][:, None] > 0.5
      rows.append(jnp.max(jnp.where(mask, hb, -jnp.inf), axis=0))
    o_ref[...] = jnp.stack(rows)

  return pl.pallas_call(
      kern,
      out_shape=jax.ShapeDtypeStruct((B, D), jnp.float32),
  )(h, oh)


def _pad_edges(idx_row, fill):
  return jnp.concatenate(
      [idx_row, jnp.full((E_PAD - E,), fill, dtype=jnp.int32)])


def kernel(graph_x, bert_x, edge_index, graph_x_batch, x, x_batch,
           rootindex, raw_edge_index, W_conv1, b_conv1, W_c0, b_c0,
           W_c1, b_c1, W_c2, b_c2, W_lin1, b_lin1, W_lin2, b_lin2,
           W_lin5, b_lin5, prelu_a):
  f32 = jnp.float32
  pad_rows = NPAD - N

  src_g = _pad_edges(edge_index[0], 0)
  dst_g = _pad_edges(edge_index[1], DUMP_ROW)
  src_x = _pad_edges(raw_edge_index[0], 0)
  dst_x = _pad_edges(raw_edge_index[1], DUMP_ROW)

  onesD = jnp.ones((CHUNK, D), f32)
  zerosD = jnp.zeros((ROWS_PER_TILE, D), f32)

  batch_g = jnp.pad(graph_x_batch, (0, pad_rows), constant_values=-1)[None]
  batch_x = jnp.pad(x_batch, (0, pad_rows), constant_values=-1)[None]

  degg, degx = _sc_degrees(dst_g, dst_x, onesD, zerosD)
  dinv_g_b = _dinv_bcast(degg[0], degg[1])
  dinv_x_b = _dinv_bcast(degx[0], degx[1])

  oh_g, _ = _onehot_and_invcnt(batch_g)
  oh_x, invcnt_x = _onehot_and_invcnt(batch_x)
  rsel = _rootsel(rootindex[None].astype(jnp.int32))

  bias_row = lambda v: v[None]
  alpha_arr = prelu_a.reshape(1, 1)

  y_g = _mm(bert_x, W_conv1, row_scale=dinv_g_b, bm=2000, bn=128, bk=128)
  agg_g = _sc_aggregate(y_g, src_g, dst_g, zerosD)
  h_g = _conv_epilogue(agg_g[0], agg_g[1], y_g, dinv_g_b, bias_row(b_conv1))
  h_pool = _segment_max(h_g, oh_g)

  mean_x = _mm(oh_x, x, row_scale=invcnt_x, bm=16, bn=512, bk=128)
  root_x = _mm(rsel, x, bm=16, bn=512, bk=128)
  cat1 = jnp.concatenate([mean_x, root_x], axis=1)
  new_x = _mm(cat1, W_lin1, bias=bias_row(b_lin1), act="prelu",
              alpha=alpha_arr, bm=16, bn=256, bk=128)
  new_x = _mm(new_x, W_lin2, bias=bias_row(b_lin2), act="prelu",
              alpha=alpha_arr, bm=16, bn=128, bk=256)

  Wc0p = jnp.pad(W_c0, ((0, DRAW_PAD - DRAW), (0, 0)))
  h = None
  pools = []
  y = _mm(x, Wc0p, row_scale=dinv_x_b, bm=2000, bn=128, bk=512)
  for Wc, bc in ((W_c0, b_c0), (W_c1, b_c1), (W_c2, b_c2)):
    if h is not None:
      y = _mm(h, Wc, row_scale=dinv_x_b, bm=1264, bn=128, bk=128)
    agg = _sc_aggregate(y, src_x, dst_x, zerosD)
    h = _conv_epilogue(agg[0], agg[1], y, dinv_x_b, bias_row(bc))
    pools.append(_mm(oh_x, h, row_scale=invcnt_x, bm=16, bn=128, bk=128))

  cat = jnp.concatenate(pools + [new_x, h_pool], axis=1)
  W5p = jnp.pad(W_lin5, ((0, 0), (0, 128 - NCLS)))
  b5p = jnp.pad(b_lin5, (0, 128 - NCLS))
  out = _mm(cat, W5p, bias=bias_row(b5p), act="log_softmax",
            n_valid=NCLS, bm=16, bn=128, bk=128 * 5)
  return out[:, :NCLS]


# speedup vs baseline: 1.2928x; 1.2928x over previous
import jax
import jax.numpy as jnp
from jax import lax
from jax.experimental import pallas as pl
from jax.experimental.pallas import tpu as pltpu
from jax.experimental.pallas import tpu_sc as plsc

N = 10000
NPAD = 10112
B = 16
E = 320000
D = 128
DRAW = 5000
DRAW_PAD = 5120
NCLS = 4

NC, NS = 2, 16
NW = NC * NS
CHUNK = 128
EDGES_PER_TILE = 10112
NCHUNK = EDGES_PER_TILE // CHUNK
E_PAD = EDGES_PER_TILE * NW
ROWS_PER_TILE = NPAD // NS
DUMP_ROW = N

import functools


@functools.cache
def _get_mesh():
  return plsc.VectorSubcoreMesh(core_axis_name="c", subcore_axis_name="s",
                                num_cores=NC, num_subcores=NS)


def _sc_degrees(dst_g, dst_x, ones_hbm, zeros_hbm):
  def body(dstg_hbm, dstx_hbm, ones_h, zeros_h, outg, outx,
           acc, dst_v, ones_v):
    cid = lax.axis_index("c")
    sid = lax.axis_index("s")
    tile = cid * NS + sid
    sl = pl.ds(sid * ROWS_PER_TILE, ROWS_PER_TILE)
    pltpu.sync_copy(ones_h, ones_v)

    def one_set(dst_hbm, out):
      pltpu.sync_copy(zeros_h, acc.at[sl])
      plsc.subcore_barrier()

      def loop(i, carry):
        base = pl.multiple_of(tile * EDGES_PER_TILE + i * CHUNK, CHUNK)
        pltpu.sync_copy(dst_hbm.at[pl.ds(base, CHUNK)], dst_v)
        pltpu.sync_copy(ones_v, acc.at[dst_v], add=True)
        return carry

      lax.fori_loop(0, NCHUNK, loop, 0)
      plsc.subcore_barrier()
      pltpu.sync_copy(acc.at[sl], out.at[cid, sl])

    one_set(dstg_hbm, outg)
    one_set(dstx_hbm, outx)

  f = pl.kernel(
      body,
      out_type=(jax.ShapeDtypeStruct((NC, NPAD, D), jnp.float32),
                jax.ShapeDtypeStruct((NC, NPAD, D), jnp.float32)),
      mesh=_get_mesh(),
      scratch_types=[
          pltpu.VMEM_SHARED((NPAD, D), jnp.float32),
          pltpu.VMEM((CHUNK,), jnp.int32),
          pltpu.VMEM((CHUNK, D), jnp.float32),
      ],
  )
  return f(dst_g, dst_x, ones_hbm, zeros_hbm)


def _sc_aggregate(y, src, dst, zeros_hbm):
  def body(y_hbm, src_hbm, dst_hbm, zeros_h, out, acc,
           s0, d0, r0, s1, d1, r1, sem0, sem1):
    cid = lax.axis_index("c")
    sid = lax.axis_index("s")
    tile = cid * NS + sid
    sl = pl.ds(sid * ROWS_PER_TILE, ROWS_PER_TILE)
    pltpu.sync_copy(zeros_h, acc.at[sl])
    plsc.subcore_barrier()
    base0 = tile * EDGES_PER_TILE

    def loop(i, carry):
      off = pl.multiple_of(base0 + i * CHUNK, CHUNK)
      pltpu.sync_copy(src_hbm.at[pl.ds(off, CHUNK)], s0)
      pltpu.sync_copy(dst_hbm.at[pl.ds(off, CHUNK)], d0)
      pltpu.async_copy(y_hbm.at[s0], r0, sem0).wait()
      pltpu.sync_copy(r0, acc.at[d0], add=True)
      return carry

    lax.fori_loop(0, NCHUNK, loop, 0)
    plsc.subcore_barrier()
    pltpu.sync_copy(acc.at[sl], out.at[cid, sl])

  f = pl.kernel(
      body,
      out_type=jax.ShapeDtypeStruct((NC, NPAD, D), jnp.float32),
      mesh=_get_mesh(),
      scratch_types=[
          pltpu.VMEM_SHARED((NPAD, D), jnp.float32),
          pltpu.VMEM((CHUNK,), jnp.int32),
          pltpu.VMEM((CHUNK,), jnp.int32),
          pltpu.VMEM((CHUNK, D), jnp.float32),
          pltpu.VMEM((CHUNK,), jnp.int32),
          pltpu.VMEM((CHUNK,), jnp.int32),
          pltpu.VMEM((CHUNK, D), jnp.float32),
          pltpu.SemaphoreType.DMA,
          pltpu.SemaphoreType.DMA,
      ],
  )
  return f(y, src, dst, zeros_hbm)


def _mm(a, b, *, bm, bn, bk, bias=None, row_scale=None, act=None,
        alpha=None, n_valid=None):
  M, Ka = a.shape
  Kb, Nn = b.shape
  K = max(Ka, Kb)
  nk = -(-K // bk)
  assert M % bm == 0
  grid = (M // bm, -(-Nn // bn), nk)

  def kern(*refs):
    k = pl.program_id(2)
    it = iter(refs)
    alpha_ref = next(it) if act == "prelu" else None
    a_ref = next(it)
    b_ref = next(it)
    rs_ref = next(it) if row_scale is not None else None
    bias_ref = next(it) if bias is not None else None
    out_ref = next(it)

    @pl.when(k == 0)
    def _():
      out_ref[...] = jnp.zeros_like(out_ref)

    ab = a_ref[...]
    bb = b_ref[...]
    if Ka < nk * bk:
      rem = Ka - (nk - 1) * bk
      lim = jnp.where(k == nk - 1, rem, bk)
      col = lax.broadcasted_iota(jnp.int32, ab.shape, 1)
      ab = jnp.where(col < lim, ab, 0.0)
    if Kb < nk * bk:
      rem = Kb - (nk - 1) * bk
      lim = jnp.where(k == nk - 1, rem, bk)
      row = lax.broadcasted_iota(jnp.int32, bb.shape, 0)
      bb = jnp.where(row < lim, bb, 0.0)

    out_ref[...] += jnp.dot(ab, bb, preferred_element_type=jnp.float32)

    @pl.when(k == nk - 1)
    def _():
      acc = out_ref[...]
      if rs_ref is not None:
        acc = acc * rs_ref[...][:, :1]
      if bias_ref is not None:
        acc = acc + bias_ref[...]
      if act == "prelu":
        al = alpha_ref[0, 0]
        acc = jnp.where(acc >= 0, acc, al * acc)
      elif act == "log_softmax":
        colv = lax.broadcasted_iota(jnp.int32, acc.shape, 1)
        valid = colv < n_valid
        z = jnp.where(valid, acc, -jnp.inf)
        m = jnp.max(z, axis=1, keepdims=True)
        e = jnp.where(valid, jnp.exp(z - m), 0.0)
        lse = m + jnp.log(jnp.sum(e, axis=1, keepdims=True))
        acc = z - lse
      out_ref[...] = acc

  in_specs = []
  ops = []
  if act == "prelu":
    in_specs.append(pl.BlockSpec(memory_space=pltpu.SMEM))
    ops.append(alpha)
  in_specs += [
      pl.BlockSpec((bm, bk), lambda i, j, k: (i, k)),
      pl.BlockSpec((bk, bn), lambda i, j, k: (k, j)),
  ]
  ops += [a, b]
  if row_scale is not None:
    in_specs.append(pl.BlockSpec((bm, 128), lambda i, j, k: (i, 0)))
    ops.append(row_scale)
  if bias is not None:
    in_specs.append(pl.BlockSpec((1, bn), lambda i, j, k: (0, j)))
    ops.append(bias)

  return pl.pallas_call(
      kern,
      grid=grid,
      in_specs=in_specs,
      out_specs=pl.BlockSpec((bm, bn), lambda i, j, k: (i, j)),
      out_shape=jax.ShapeDtypeStruct((M, Nn), jnp.float32),
      compiler_params=pltpu.CompilerParams(
          dimension_semantics=("parallel", "parallel", "arbitrary")),
  )(*ops)


def _onehot_and_invcnt(batch_pad):
  ncol = NPAD // 128

  def kern(ids_ref, oh_ref, cnt_ref):
    j = pl.program_id(0)
    ids = ids_ref[...]
    row = lax.broadcasted_iota(jnp.int32, (B, 128), 0)
    oh = (ids == row).astype(jnp.float32)
    oh_ref[...] = oh

    @pl.when(j == 0)
    def _():
      cnt_ref[...] = jnp.zeros_like(cnt_ref)

    cnt_ref[...] += jnp.sum(oh, axis=1, keepdims=True)

    @pl.when(j == ncol - 1)
    def _():
      cnt_ref[...] = 1.0 / jnp.maximum(cnt_ref[...], 1.0)

  return pl.pallas_call(
      kern,
      grid=(ncol,),
      in_specs=[pl.BlockSpec((1, 128), lambda j: (0, j))],
      out_specs=[pl.BlockSpec((B, 128), lambda j: (0, j)),
                 pl.BlockSpec((B, 128), lambda j: (0, 0))],
      out_shape=[jax.ShapeDtypeStruct((B, NPAD), jnp.float32),
                 jax.ShapeDtypeStruct((B, 128), jnp.float32)],
      compiler_params=pltpu.CompilerParams(
          dimension_semantics=("arbitrary",)),
  )(batch_pad)


def _rootsel(rootindex_2d):
  ncol = NPAD // 128

  def kern(root_ref, sel_ref):
    j = pl.program_id(0)
    roots = jnp.stack([root_ref[0, b] for b in range(B)])
    col = lax.broadcasted_iota(jnp.int32, (B, 128), 1) + j * 128
    sel_ref[...] = (col == roots[:, None]).astype(jnp.float32)

  return pl.pallas_call(
      kern,
      grid=(ncol,),
      in_specs=[pl.BlockSpec(memory_space=pltpu.SMEM)],
      out_specs=pl.BlockSpec((B, 128), lambda j: (0, j)),
      out_shape=jax.ShapeDtypeStruct((B, NPAD), jnp.float32),
  )(rootindex_2d)


def _dinv_bcast(p0, p1):
  bm = 1264

  def kern(a_ref, b_ref, o_ref):
    o_ref[...] = lax.rsqrt(a_ref[...] + b_ref[...] + 1.0)

  return pl.pallas_call(
      kern,
      grid=(NPAD // bm,),
      in_specs=[pl.BlockSpec((bm, D), lambda i: (i, 0))] * 2,
      out_specs=pl.BlockSpec((bm, D), lambda i: (i, 0)),
      out_shape=jax.ShapeDtypeStruct((NPAD, D), jnp.float32),
  )(p0, p1)


def _conv_epilogue(p0, p1, y, dinv_b, bias):
  bm = 1264

  def kern(p0_ref, p1_ref, y_ref, d_ref, b_ref, o_ref):
    i = pl.program_id(0)
    s = (p0_ref[...] + p1_ref[...] + y_ref[...]) * d_ref[...]
    v = jnp.maximum(s + b_ref[...], 0.0)
    rowg = lax.broadcasted_iota(jnp.int32, v.shape, 0) + i * bm
    o_ref[...] = jnp.where(rowg < N, v, 0.0)

  return pl.pallas_call(
      kern,
      grid=(NPAD // bm,),
      in_specs=[pl.BlockSpec((bm, D), lambda i: (i, 0))] * 4 +
               [pl.BlockSpec((1, D), lambda i: (0, 0))],
      out_specs=pl.BlockSpec((bm, D), lambda i: (i, 0)),
      out_shape=jax.ShapeDtypeStruct((NPAD, D), jnp.float32),
  )(p0, p1, y, dinv_b, bias)


def _segment_max(h, oh):
  def kern(h_ref, oh_ref, o_ref):
    hb = h_ref[...]
    rows = []
    for b in range(B):
      mask = oh_ref[b, :][:, None] > 0.5
      rows.append(jnp.max(jnp.where(mask, hb, -jnp.inf), axis=0))
    o_ref[...] = jnp.stack(rows)

  return pl.pallas_call(
      kern,
      out_shape=jax.ShapeDtypeStruct((B, D), jnp.float32),
  )(h, oh)


def _pad_edges(idx_row, fill):
  return jnp.concatenate(
      [idx_row, jnp.full((E_PAD - E,), fill, dtype=jnp.int32)])


def kernel(graph_x, bert_x, edge_index, graph_x_batch, x, x_batch,
           rootindex, raw_edge_index, W_conv1, b_conv1, W_c0, b_c0,
           W_c1, b_c1, W_c2, b_c2, W_lin1, b_lin1, W_lin2, b_lin2,
           W_lin5, b_lin5, prelu_a):
  f32 = jnp.float32
  pad_rows = NPAD - N

  src_g = _pad_edges(edge_index[0], 0)
  dst_g = _pad_edges(edge_index[1], DUMP_ROW)
  src_x = _pad_edges(raw_edge_index[0], 0)
  dst_x = _pad_edges(raw_edge_index[1], DUMP_ROW)

  onesD = jnp.ones((CHUNK, D), f32)
  zerosD = jnp.zeros((ROWS_PER_TILE, D), f32)

  batch_g = jnp.pad(graph_x_batch, (0, pad_rows), constant_values=-1)[None]
  batch_x = jnp.pad(x_batch, (0, pad_rows), constant_values=-1)[None]

  degg, degx = _sc_degrees(dst_g, dst_x, onesD, zerosD)
  dinv_g_b = _dinv_bcast(degg[0], degg[1])
  dinv_x_b = _dinv_bcast(degx[0], degx[1])

  oh_g, _ = _onehot_and_invcnt(batch_g)
  oh_x, invcnt_x = _onehot_and_invcnt(batch_x)
  rsel = _rootsel(rootindex[None].astype(jnp.int32))

  bias_row = lambda v: v[None]
  alpha_arr = prelu_a.reshape(1, 1)

  y_g = _mm(bert_x, W_conv1, row_scale=dinv_g_b, bm=2000, bn=128, bk=128)
  agg_g = _sc_aggregate(y_g, src_g, dst_g, zerosD)
  h_g = _conv_epilogue(agg_g[0], agg_g[1], y_g, dinv_g_b, bias_row(b_conv1))
  h_pool = _segment_max(h_g, oh_g)

  sel2 = jnp.concatenate([oh_x, rsel], axis=0)
  rs2 = jnp.concatenate([invcnt_x, jnp.ones((B, 128), f32)], axis=0)
  mr = _mm(sel2, x, row_scale=rs2, bm=32, bn=512, bk=128)
  cat1 = jnp.concatenate([mr[:B], mr[B:]], axis=1)
  new_x = _mm(cat1, W_lin1, bias=bias_row(b_lin1), act="prelu",
              alpha=alpha_arr, bm=16, bn=256, bk=128)
  new_x = _mm(new_x, W_lin2, bias=bias_row(b_lin2), act="prelu",
              alpha=alpha_arr, bm=16, bn=128, bk=256)

  Wc0p = jnp.pad(W_c0, ((0, DRAW_PAD - DRAW), (0, 0)))
  h = None
  pools = []
  y = _mm(x, Wc0p, row_scale=dinv_x_b, bm=2000, bn=128, bk=512)
  for Wc, bc in ((W_c0, b_c0), (W_c1, b_c1), (W_c2, b_c2)):
    if h is not None:
      y = _mm(h, Wc, row_scale=dinv_x_b, bm=1264, bn=128, bk=128)
    agg = _sc_aggregate(y, src_x, dst_x, zerosD)
    h = _conv_epilogue(agg[0], agg[1], y, dinv_x_b, bias_row(bc))
    pools.append(_mm(oh_x, h, row_scale=invcnt_x, bm=16, bn=128, bk=128))

  cat = jnp.concatenate(pools + [new_x, h_pool], axis=1)
  W5p = jnp.pad(W_lin5, ((0, 0), (0, 128 - NCLS)))
  b5p = jnp.pad(b_lin5, (0, 128 - NCLS))
  out = _mm(cat, W5p, bias=bias_row(b5p), act="log_softmax",
            n_valid=NCLS, bm=16, bn=128, bk=128 * 5)
  return out[:, :NCLS]


# speedup vs baseline: 1.3493x; 1.0437x over previous
import jax
import jax.numpy as jnp
from jax import lax
from jax.experimental import pallas as pl
from jax.experimental.pallas import tpu as pltpu
from jax.experimental.pallas import tpu_sc as plsc

N = 10000
NPAD = 10112
B = 16
E = 320000
D = 128
DRAW = 5000
DRAW_PAD = 5120
NCLS = 4

NC, NS = 2, 16
NW = NC * NS
CHUNK = 128
EDGES_PER_TILE = 10112
NCHUNK = EDGES_PER_TILE // CHUNK
E_PAD = EDGES_PER_TILE * NW
ROWS_PER_TILE = NPAD // NS
DUMP_ROW = N

import functools


@functools.cache
def _get_mesh():
  return plsc.VectorSubcoreMesh(core_axis_name="c", subcore_axis_name="s",
                                num_cores=NC, num_subcores=NS)


def _sc_degrees(dst_g, dst_x, ones_hbm, zeros_hbm):
  def body(dstg_hbm, dstx_hbm, ones_h, zeros_h, outg, outx,
           acc, dst_v, ones_v):
    cid = lax.axis_index("c")
    sid = lax.axis_index("s")
    tile = cid * NS + sid
    sl = pl.ds(sid * ROWS_PER_TILE, ROWS_PER_TILE)
    pltpu.sync_copy(ones_h, ones_v)

    def one_set(dst_hbm, out):
      pltpu.sync_copy(zeros_h, acc.at[sl])
      plsc.subcore_barrier()

      def loop(i, carry):
        base = pl.multiple_of(tile * EDGES_PER_TILE + i * CHUNK, CHUNK)
        pltpu.sync_copy(dst_hbm.at[pl.ds(base, CHUNK)], dst_v)
        pltpu.sync_copy(ones_v, acc.at[dst_v], add=True)
        return carry

      lax.fori_loop(0, NCHUNK, loop, 0)
      plsc.subcore_barrier()
      pltpu.sync_copy(acc.at[sl], out.at[cid, sl])

    one_set(dstg_hbm, outg)
    one_set(dstx_hbm, outx)

  f = pl.kernel(
      body,
      out_type=(jax.ShapeDtypeStruct((NC, NPAD, D), jnp.float32),
                jax.ShapeDtypeStruct((NC, NPAD, D), jnp.float32)),
      mesh=_get_mesh(),
      scratch_types=[
          pltpu.VMEM_SHARED((NPAD, D), jnp.float32),
          pltpu.VMEM((CHUNK,), jnp.int32),
          pltpu.VMEM((CHUNK, D), jnp.float32),
      ],
  )
  return f(dst_g, dst_x, ones_hbm, zeros_hbm)


def _sc_aggregate(y, idx3, zeros_hbm):
  HALF = CHUNK // 2

  def body(y_hbm, idx_hbm, zeros_h, out, acc, ii, r0, sem0, sem1):
    cid = lax.axis_index("c")
    sid = lax.axis_index("s")
    tile = cid * NS + sid
    sl = pl.ds(sid * ROWS_PER_TILE, ROWS_PER_TILE)
    pltpu.sync_copy(zeros_h, acc.at[sl])
    plsc.subcore_barrier()
    cbase = tile * NCHUNK

    def loop(i, carry):
      pltpu.sync_copy(idx_hbm.at[cbase + i], ii)
      a = pltpu.async_copy(y_hbm.at[ii.at[0, pl.ds(0, HALF)]],
                           r0.at[pl.ds(0, HALF)], sem0)
      b = pltpu.async_copy(y_hbm.at[ii.at[0, pl.ds(HALF, HALF)]],
                           r0.at[pl.ds(HALF, HALF)], sem1)
      a.wait()
      b.wait()
      pltpu.sync_copy(r0, acc.at[ii.at[1]], add=True)
      return carry

    lax.fori_loop(0, NCHUNK, loop, 0)
    plsc.subcore_barrier()
    pltpu.sync_copy(acc.at[sl], out.at[cid, sl])

  f = pl.kernel(
      body,
      out_type=jax.ShapeDtypeStruct((NC, NPAD, D), jnp.float32),
      mesh=_get_mesh(),
      scratch_types=[
          pltpu.VMEM_SHARED((NPAD, D), jnp.float32),
          pltpu.VMEM((2, CHUNK), jnp.int32),
          pltpu.VMEM((CHUNK, D), jnp.float32),
          pltpu.SemaphoreType.DMA,
          pltpu.SemaphoreType.DMA,
      ],
  )
  return f(y, idx3, zeros_hbm)


def _mm(a, b, *, bm, bn, bk, bias=None, row_scale=None, act=None,
        alpha=None, n_valid=None):
  M, Ka = a.shape
  Kb, Nn = b.shape
  K = max(Ka, Kb)
  nk = -(-K // bk)
  assert M % bm == 0
  grid = (M // bm, -(-Nn // bn), nk)

  def kern(*refs):
    k = pl.program_id(2)
    it = iter(refs)
    alpha_ref = next(it) if act == "prelu" else None
    a_ref = next(it)
    b_ref = next(it)
    rs_ref = next(it) if row_scale is not None else None
    bias_ref = next(it) if bias is not None else None
    out_ref = next(it)

    @pl.when(k == 0)
    def _():
      out_ref[...] = jnp.zeros_like(out_ref)

    ab = a_ref[...]
    bb = b_ref[...]
    if Ka < nk * bk:
      rem = Ka - (nk - 1) * bk
      lim = jnp.where(k == nk - 1, rem, bk)
      col = lax.broadcasted_iota(jnp.int32, ab.shape, 1)
      ab = jnp.where(col < lim, ab, 0.0)
    if Kb < nk * bk:
      rem = Kb - (nk - 1) * bk
      lim = jnp.where(k == nk - 1, rem, bk)
      row = lax.broadcasted_iota(jnp.int32, bb.shape, 0)
      bb = jnp.where(row < lim, bb, 0.0)

    out_ref[...] += jnp.dot(ab, bb, preferred_element_type=jnp.float32)

    @pl.when(k == nk - 1)
    def _():
      acc = out_ref[...]
      if rs_ref is not None:
        acc = acc * rs_ref[...][:, :1]
      if bias_ref is not None:
        acc = acc + bias_ref[...]
      if act == "prelu":
        al = alpha_ref[0, 0]
        acc = jnp.where(acc >= 0, acc, al * acc)
      elif act == "log_softmax":
        colv = lax.broadcasted_iota(jnp.int32, acc.shape, 1)
        valid = colv < n_valid
        z = jnp.where(valid, acc, -jnp.inf)
        m = jnp.max(z, axis=1, keepdims=True)
        e = jnp.where(valid, jnp.exp(z - m), 0.0)
        lse = m + jnp.log(jnp.sum(e, axis=1, keepdims=True))
        acc = z - lse
      out_ref[...] = acc

  in_specs = []
  ops = []
  if act == "prelu":
    in_specs.append(pl.BlockSpec(memory_space=pltpu.SMEM))
    ops.append(alpha)
  in_specs += [
      pl.BlockSpec((bm, bk), lambda i, j, k: (i, k)),
      pl.BlockSpec((bk, bn), lambda i, j, k: (k, j)),
  ]
  ops += [a, b]
  if row_scale is not None:
    in_specs.append(pl.BlockSpec((bm, 128), lambda i, j, k: (i, 0)))
    ops.append(row_scale)
  if bias is not None:
    in_specs.append(pl.BlockSpec((1, bn), lambda i, j, k: (0, j)))
    ops.append(bias)

  return pl.pallas_call(
      kern,
      grid=grid,
      in_specs=in_specs,
      out_specs=pl.BlockSpec((bm, bn), lambda i, j, k: (i, j)),
      out_shape=jax.ShapeDtypeStruct((M, Nn), jnp.float32),
      compiler_params=pltpu.CompilerParams(
          dimension_semantics=("parallel", "parallel", "arbitrary")),
  )(*ops)


def _onehot_and_invcnt(batch_pad):
  ncol = NPAD // 128

  def kern(ids_ref, oh_ref, cnt_ref):
    j = pl.program_id(0)
    ids = ids_ref[...]
    row = lax.broadcasted_iota(jnp.int32, (B, 128), 0)
    oh = (ids == row).astype(jnp.float32)
    oh_ref[...] = oh

    @pl.when(j == 0)
    def _():
      cnt_ref[...] = jnp.zeros_like(cnt_ref)

    cnt_ref[...] += jnp.sum(oh, axis=1, keepdims=True)

    @pl.when(j == ncol - 1)
    def _():
      cnt_ref[...] = 1.0 / jnp.maximum(cnt_ref[...], 1.0)

  return pl.pallas_call(
      kern,
      grid=(ncol,),
      in_specs=[pl.BlockSpec((1, 128), lambda j: (0, j))],
      out_specs=[pl.BlockSpec((B, 128), lambda j: (0, j)),
                 pl.BlockSpec((B, 128), lambda j: (0, 0))],
      out_shape=[jax.ShapeDtypeStruct((B, NPAD), jnp.float32),
                 jax.ShapeDtypeStruct((B, 128), jnp.float32)],
      compiler_params=pltpu.CompilerParams(
          dimension_semantics=("arbitrary",)),
  )(batch_pad)


def _rootsel(rootindex_2d):
  ncol = NPAD // 128

  def kern(root_ref, sel_ref):
    j = pl.program_id(0)
    roots = jnp.stack([root_ref[0, b] for b in range(B)])
    col = lax.broadcasted_iota(jnp.int32, (B, 128), 1) + j * 128
    sel_ref[...] = (col == roots[:, None]).astype(jnp.float32)

  return pl.pallas_call(
      kern,
      grid=(ncol,),
      in_specs=[pl.BlockSpec(memory_space=pltpu.SMEM)],
      out_specs=pl.BlockSpec((B, 128), lambda j: (0, j)),
      out_shape=jax.ShapeDtypeStruct((B, NPAD), jnp.float32),
  )(rootindex_2d)


def _dinv_bcast(p0, p1):
  bm = 1264

  def kern(a_ref, b_ref, o_ref):
    o_ref[...] = lax.rsqrt(a_ref[...] + b_ref[...] + 1.0)

  return pl.pallas_call(
      kern,
      grid=(NPAD // bm,),
      in_specs=[pl.BlockSpec((bm, D), lambda i: (i, 0))] * 2,
      out_specs=pl.BlockSpec((bm, D), lambda i: (i, 0)),
      out_shape=jax.ShapeDtypeStruct((NPAD, D), jnp.float32),
  )(p0, p1)


def _conv_epilogue(p0, p1, y, dinv_b, bias):
  bm = 1264

  def kern(p0_ref, p1_ref, y_ref, d_ref, b_ref, o_ref):
    i = pl.program_id(0)
    s = (p0_ref[...] + p1_ref[...] + y_ref[...]) * d_ref[...]
    v = jnp.maximum(s + b_ref[...], 0.0)
    rowg = lax.broadcasted_iota(jnp.int32, v.shape, 0) + i * bm
    o_ref[...] = jnp.where(rowg < N, v, 0.0)

  return pl.pallas_call(
      kern,
      grid=(NPAD // bm,),
      in_specs=[pl.BlockSpec((bm, D), lambda i: (i, 0))] * 4 +
               [pl.BlockSpec((1, D), lambda i: (0, 0))],
      out_specs=pl.BlockSpec((bm, D), lambda i: (i, 0)),
      out_shape=jax.ShapeDtypeStruct((NPAD, D), jnp.float32),
  )(p0, p1, y, dinv_b, bias)


def _segment_max(h, oh):
  def kern(h_ref, oh_ref, o_ref):
    hb = h_ref[...]
    rows = []
    for b in range(B):
      mask = oh_ref[b, :][:, None] > 0.5
      rows.append(jnp.max(jnp.where(mask, hb, -jnp.inf), axis=0))
    o_ref[...] = jnp.stack(rows)

  return pl.pallas_call(
      kern,
      out_shape=jax.ShapeDtypeStruct((B, D), jnp.float32),
  )(h, oh)


def _pad_edges(idx_row, fill):
  return jnp.concatenate(
      [idx_row, jnp.full((E_PAD - E,), fill, dtype=jnp.int32)])


def kernel(graph_x, bert_x, edge_index, graph_x_batch, x, x_batch,
           rootindex, raw_edge_index, W_conv1, b_conv1, W_c0, b_c0,
           W_c1, b_c1, W_c2, b_c2, W_lin1, b_lin1, W_lin2, b_lin2,
           W_lin5, b_lin5, prelu_a):
  f32 = jnp.float32
  pad_rows = NPAD - N

  src_g = _pad_edges(edge_index[0], 0)
  dst_g = _pad_edges(edge_index[1], DUMP_ROW)
  src_x = _pad_edges(raw_edge_index[0], 0)
  dst_x = _pad_edges(raw_edge_index[1], DUMP_ROW)
  idx3_g = jnp.stack([src_g.reshape(-1, CHUNK), dst_g.reshape(-1, CHUNK)],
                     axis=1)
  idx3_x = jnp.stack([src_x.reshape(-1, CHUNK), dst_x.reshape(-1, CHUNK)],
                     axis=1)

  onesD = jnp.ones((CHUNK, D), f32)
  zerosD = jnp.zeros((ROWS_PER_TILE, D), f32)

  batch_g = jnp.pad(graph_x_batch, (0, pad_rows), constant_values=-1)[None]
  batch_x = jnp.pad(x_batch, (0, pad_rows), constant_values=-1)[None]

  degg, degx = _sc_degrees(dst_g, dst_x, onesD, zerosD)
  dinv_g_b = _dinv_bcast(degg[0], degg[1])
  dinv_x_b = _dinv_bcast(degx[0], degx[1])

  oh_g, _ = _onehot_and_invcnt(batch_g)
  oh_x, invcnt_x = _onehot_and_invcnt(batch_x)
  rsel = _rootsel(rootindex[None].astype(jnp.int32))

  bias_row = lambda v: v[None]
  alpha_arr = prelu_a.reshape(1, 1)

  y_g = _mm(bert_x, W_conv1, row_scale=dinv_g_b, bm=2000, bn=128, bk=128)
  agg_g = _sc_aggregate(y_g, idx3_g, zerosD)
  h_g = _conv_epilogue(agg_g[0], agg_g[1], y_g, dinv_g_b, bias_row(b_conv1))
  h_pool = _segment_max(h_g, oh_g)

  sel2 = jnp.concatenate([oh_x, rsel], axis=0)
  rs2 = jnp.concatenate([invcnt_x, jnp.ones((B, 128), f32)], axis=0)
  mr = _mm(sel2, x, row_scale=rs2, bm=32, bn=512, bk=128)
  cat1 = jnp.concatenate([mr[:B], mr[B:]], axis=1)
  new_x = _mm(cat1, W_lin1, bias=bias_row(b_lin1), act="prelu",
              alpha=alpha_arr, bm=16, bn=256, bk=128)
  new_x = _mm(new_x, W_lin2, bias=bias_row(b_lin2), act="prelu",
              alpha=alpha_arr, bm=16, bn=128, bk=256)

  Wc0p = jnp.pad(W_c0, ((0, DRAW_PAD - DRAW), (0, 0)))
  h = None
  pools = []
  y = _mm(x, Wc0p, row_scale=dinv_x_b, bm=2000, bn=128, bk=512)
  for Wc, bc in ((W_c0, b_c0), (W_c1, b_c1), (W_c2, b_c2)):
    if h is not None:
      y = _mm(h, Wc, row_scale=dinv_x_b, bm=1264, bn=128, bk=128)
    agg = _sc_aggregate(y, idx3_x, zerosD)
    h = _conv_epilogue(agg[0], agg[1], y, dinv_x_b, bias_row(bc))
    pools.append(_mm(oh_x, h, row_scale=invcnt_x, bm=16, bn=128, bk=128))

  cat = jnp.concatenate(pools + [new_x, h_pool], axis=1)
  W5p = jnp.pad(W_lin5, ((0, 0), (0, 128 - NCLS)))
  b5p = jnp.pad(b_lin5, (0, 128 - NCLS))
  out = _mm(cat, W5p, bias=bias_row(b5p), act="log_softmax",
            n_valid=NCLS, bm=16, bn=128, bk=128 * 5)
  return out[:, :NCLS]


# speedup vs baseline: 1.3495x; 1.0002x over previous
import jax
import jax.numpy as jnp
from jax import lax
from jax.experimental import pallas as pl
from jax.experimental.pallas import tpu as pltpu
from jax.experimental.pallas import tpu_sc as plsc

N = 10000
NPAD = 10112
B = 16
E = 320000
D = 128
DRAW = 5000
DRAW_PAD = 5120
NCLS = 4

NC, NS = 2, 16
NW = NC * NS
CHUNK = 128
EDGES_PER_TILE = 10112
NCHUNK = EDGES_PER_TILE // CHUNK
E_PAD = EDGES_PER_TILE * NW
ROWS_PER_TILE = NPAD // NS
DUMP_ROW = N

import functools


@functools.cache
def _get_mesh():
  return plsc.VectorSubcoreMesh(core_axis_name="c", subcore_axis_name="s",
                                num_cores=NC, num_subcores=NS)


def _sc_degrees(dst_g, dst_x, ones_hbm, zeros_hbm):
  def body(dstg_hbm, dstx_hbm, ones_h, zeros_h, outg, outx,
           acc, dst_v, ones_v):
    cid = lax.axis_index("c")
    sid = lax.axis_index("s")
    tile = cid * NS + sid
    sl = pl.ds(sid * ROWS_PER_TILE, ROWS_PER_TILE)
    pltpu.sync_copy(ones_h, ones_v)

    def one_set(dst_hbm, out):
      pltpu.sync_copy(zeros_h, acc.at[sl])
      plsc.subcore_barrier()

      def loop(i, carry):
        base = pl.multiple_of(tile * EDGES_PER_TILE + i * CHUNK, CHUNK)
        pltpu.sync_copy(dst_hbm.at[pl.ds(base, CHUNK)], dst_v)
        pltpu.sync_copy(ones_v, acc.at[dst_v], add=True)
        return carry

      lax.fori_loop(0, NCHUNK, loop, 0)
      plsc.subcore_barrier()
      pltpu.sync_copy(acc.at[sl], out.at[cid, sl])

    one_set(dstg_hbm, outg)
    one_set(dstx_hbm, outx)

  f = pl.kernel(
      body,
      out_type=(jax.ShapeDtypeStruct((NC, NPAD, D), jnp.float32),
                jax.ShapeDtypeStruct((NC, NPAD, D), jnp.float32)),
      mesh=_get_mesh(),
      scratch_types=[
          pltpu.VMEM_SHARED((NPAD, D), jnp.float32),
          pltpu.VMEM((CHUNK,), jnp.int32),
          pltpu.VMEM((CHUNK, D), jnp.float32),
      ],
  )
  return f(dst_g, dst_x, ones_hbm, zeros_hbm)


def _sc_aggregate(y, idx3, zeros_hbm):
  NSPLIT = 2
  PART = CHUNK // NSPLIT

  def body(y_hbm, idx_hbm, zeros_h, out, acc, ii, r0, *sems):
    cid = lax.axis_index("c")
    sid = lax.axis_index("s")
    tile = cid * NS + sid
    sl = pl.ds(sid * ROWS_PER_TILE, ROWS_PER_TILE)
    pltpu.sync_copy(zeros_h, acc.at[sl])
    plsc.subcore_barrier()
    cbase = tile * NCHUNK

    def loop(i, carry):
      pltpu.sync_copy(idx_hbm.at[cbase + i], ii)
      descs = [
          pltpu.async_copy(y_hbm.at[ii.at[0, pl.ds(p * PART, PART)]],
                           r0.at[pl.ds(p * PART, PART)], sems[p])
          for p in range(NSPLIT)
      ]
      for dsc in descs:
        dsc.wait()
      pltpu.sync_copy(r0, acc.at[ii.at[1]], add=True)
      return carry

    lax.fori_loop(0, NCHUNK, loop, 0)
    plsc.subcore_barrier()
    pltpu.sync_copy(acc.at[sl], out.at[cid, sl])

  f = pl.kernel(
      body,
      out_type=jax.ShapeDtypeStruct((NC, NPAD, D), jnp.float32),
      mesh=_get_mesh(),
      scratch_types=[
          pltpu.VMEM_SHARED((NPAD, D), jnp.float32),
          pltpu.VMEM((2, CHUNK), jnp.int32),
          pltpu.VMEM((CHUNK, D), jnp.float32),
      ] + [pltpu.SemaphoreType.DMA] * NSPLIT,
  )
  return f(y, idx3, zeros_hbm)


def _mm(a, b, *, bm, bn, bk, bias=None, row_scale=None, act=None,
        alpha=None, n_valid=None):
  M, Ka = a.shape
  Kb, Nn = b.shape
  K = max(Ka, Kb)
  nk = -(-K // bk)
  assert M % bm == 0
  grid = (M // bm, -(-Nn // bn), nk)

  def kern(*refs):
    k = pl.program_id(2)
    it = iter(refs)
    alpha_ref = next(it) if act == "prelu" else None
    a_ref = next(it)
    b_ref = next(it)
    rs_ref = next(it) if row_scale is not None else None
    bias_ref = next(it) if bias is not None else None
    out_ref = next(it)

    @pl.when(k == 0)
    def _():
      out_ref[...] = jnp.zeros_like(out_ref)

    ab = a_ref[...]
    bb = b_ref[...]
    if Ka < nk * bk:
      rem = Ka - (nk - 1) * bk
      lim = jnp.where(k == nk - 1, rem, bk)
      col = lax.broadcasted_iota(jnp.int32, ab.shape, 1)
      ab = jnp.where(col < lim, ab, 0.0)
    if Kb < nk * bk:
      rem = Kb - (nk - 1) * bk
      lim = jnp.where(k == nk - 1, rem, bk)
      row = lax.broadcasted_iota(jnp.int32, bb.shape, 0)
      bb = jnp.where(row < lim, bb, 0.0)

    out_ref[...] += jnp.dot(ab, bb, preferred_element_type=jnp.float32)

    @pl.when(k == nk - 1)
    def _():
      acc = out_ref[...]
      if rs_ref is not None:
        acc = acc * rs_ref[...][:, :1]
      if bias_ref is not None:
        acc = acc + bias_ref[...]
      if act == "prelu":
        al = alpha_ref[0, 0]
        acc = jnp.where(acc >= 0, acc, al * acc)
      elif act == "log_softmax":
        colv = lax.broadcasted_iota(jnp.int32, acc.shape, 1)
        valid = colv < n_valid
        z = jnp.where(valid, acc, -jnp.inf)
        m = jnp.max(z, axis=1, keepdims=True)
        e = jnp.where(valid, jnp.exp(z - m), 0.0)
        lse = m + jnp.log(jnp.sum(e, axis=1, keepdims=True))
        acc = z - lse
      out_ref[...] = acc

  in_specs = []
  ops = []
  if act == "prelu":
    in_specs.append(pl.BlockSpec(memory_space=pltpu.SMEM))
    ops.append(alpha)
  in_specs += [
      pl.BlockSpec((bm, bk), lambda i, j, k: (i, k)),
      pl.BlockSpec((bk, bn), lambda i, j, k: (k, j)),
  ]
  ops += [a, b]
  if row_scale is not None:
    in_specs.append(pl.BlockSpec((bm, 128), lambda i, j, k: (i, 0)))
    ops.append(row_scale)
  if bias is not None:
    in_specs.append(pl.BlockSpec((1, bn), lambda i, j, k: (0, j)))
    ops.append(bias)

  return pl.pallas_call(
      kern,
      grid=grid,
      in_specs=in_specs,
      out_specs=pl.BlockSpec((bm, bn), lambda i, j, k: (i, j)),
      out_shape=jax.ShapeDtypeStruct((M, Nn), jnp.float32),
      compiler_params=pltpu.CompilerParams(
          dimension_semantics=("parallel", "parallel", "arbitrary")),
  )(*ops)


def _onehot_and_invcnt(batch_pad):
  ncol = NPAD // 128

  def kern(ids_ref, oh_ref, cnt_ref):
    j = pl.program_id(0)
    ids = ids_ref[...]
    row = lax.broadcasted_iota(jnp.int32, (B, 128), 0)
    oh = (ids == row).astype(jnp.float32)
    oh_ref[...] = oh

    @pl.when(j == 0)
    def _():
      cnt_ref[...] = jnp.zeros_like(cnt_ref)

    cnt_ref[...] += jnp.sum(oh, axis=1, keepdims=True)

    @pl.when(j == ncol - 1)
    def _():
      cnt_ref[...] = 1.0 / jnp.maximum(cnt_ref[...], 1.0)

  return pl.pallas_call(
      kern,
      grid=(ncol,),
      in_specs=[pl.BlockSpec((1, 128), lambda j: (0, j))],
      out_specs=[pl.BlockSpec((B, 128), lambda j: (0, j)),
                 pl.BlockSpec((B, 128), lambda j: (0, 0))],
      out_shape=[jax.ShapeDtypeStruct((B, NPAD), jnp.float32),
                 jax.ShapeDtypeStruct((B, 128), jnp.float32)],
      compiler_params=pltpu.CompilerParams(
          dimension_semantics=("arbitrary",)),
  )(batch_pad)


def _rootsel(rootindex_2d):
  ncol = NPAD // 128

  def kern(root_ref, sel_ref):
    j = pl.program_id(0)
    roots = jnp.stack([root_ref[0, b] for b in range(B)])
    col = lax.broadcasted_iota(jnp.int32, (B, 128), 1) + j * 128
    sel_ref[...] = (col == roots[:, None]).astype(jnp.float32)

  return pl.pallas_call(
      kern,
      grid=(ncol,),
      in_specs=[pl.BlockSpec(memory_space=pltpu.SMEM)],
      out_specs=pl.BlockSpec((B, 128), lambda j: (0, j)),
      out_shape=jax.ShapeDtypeStruct((B, NPAD), jnp.float32),
  )(rootindex_2d)


def _dinv_bcast(p0, p1):
  bm = 1264

  def kern(a_ref, b_ref, o_ref):
    o_ref[...] = lax.rsqrt(a_ref[...] + b_ref[...] + 1.0)

  return pl.pallas_call(
      kern,
      grid=(NPAD // bm,),
      in_specs=[pl.BlockSpec((bm, D), lambda i: (i, 0))] * 2,
      out_specs=pl.BlockSpec((bm, D), lambda i: (i, 0)),
      out_shape=jax.ShapeDtypeStruct((NPAD, D), jnp.float32),
  )(p0, p1)


def _conv_epilogue(p0, p1, y, dinv_b, bias):
  bm = 1264

  def kern(p0_ref, p1_ref, y_ref, d_ref, b_ref, o_ref):
    i = pl.program_id(0)
    s = (p0_ref[...] + p1_ref[...] + y_ref[...]) * d_ref[...]
    v = jnp.maximum(s + b_ref[...], 0.0)
    rowg = lax.broadcasted_iota(jnp.int32, v.shape, 0) + i * bm
    o_ref[...] = jnp.where(rowg < N, v, 0.0)

  return pl.pallas_call(
      kern,
      grid=(NPAD // bm,),
      in_specs=[pl.BlockSpec((bm, D), lambda i: (i, 0))] * 4 +
               [pl.BlockSpec((1, D), lambda i: (0, 0))],
      out_specs=pl.BlockSpec((bm, D), lambda i: (i, 0)),
      out_shape=jax.ShapeDtypeStruct((NPAD, D), jnp.float32),
  )(p0, p1, y, dinv_b, bias)


def _segment_max(h, oh):
  def kern(h_ref, oh_ref, o_ref):
    hb = h_ref[...]
    rows = []
    for b in range(B):
      mask = oh_ref[b, :][:, None] > 0.5
      rows.append(jnp.max(jnp.where(mask, hb, -jnp.inf), axis=0))
    o_ref[...] = jnp.stack(rows)

  return pl.pallas_call(
      kern,
      out_shape=jax.ShapeDtypeStruct((B, D), jnp.float32),
  )(h, oh)


def _pad_edges(idx_row, fill):
  return jnp.concatenate(
      [idx_row, jnp.full((E_PAD - E,), fill, dtype=jnp.int32)])


def kernel(graph_x, bert_x, edge_index, graph_x_batch, x, x_batch,
           rootindex, raw_edge_index, W_conv1, b_conv1, W_c0, b_c0,
           W_c1, b_c1, W_c2, b_c2, W_lin1, b_lin1, W_lin2, b_lin2,
           W_lin5, b_lin5, prelu_a):
  f32 = jnp.float32
  pad_rows = NPAD - N

  src_g = _pad_edges(edge_index[0], 0)
  dst_g = _pad_edges(edge_index[1], DUMP_ROW)
  src_x = _pad_edges(raw_edge_index[0], 0)
  dst_x = _pad_edges(raw_edge_index[1], DUMP_ROW)
  idx3_g = jnp.stack([src_g.reshape(-1, CHUNK), dst_g.reshape(-1, CHUNK)],
                     axis=1)
  idx3_x = jnp.stack([src_x.reshape(-1, CHUNK), dst_x.reshape(-1, CHUNK)],
                     axis=1)

  onesD = jnp.ones((CHUNK, D), f32)
  zerosD = jnp.zeros((ROWS_PER_TILE, D), f32)

  batch_g = jnp.pad(graph_x_batch, (0, pad_rows), constant_values=-1)[None]
  batch_x = jnp.pad(x_batch, (0, pad_rows), constant_values=-1)[None]

  degg, degx = _sc_degrees(dst_g, dst_x, onesD, zerosD)
  dinv_g_b = _dinv_bcast(degg[0], degg[1])
  dinv_x_b = _dinv_bcast(degx[0], degx[1])

  oh_g, _ = _onehot_and_invcnt(batch_g)
  oh_x, invcnt_x = _onehot_and_invcnt(batch_x)
  rsel = _rootsel(rootindex[None].astype(jnp.int32))

  bias_row = lambda v: v[None]
  alpha_arr = prelu_a.reshape(1, 1)

  y_g = _mm(bert_x, W_conv1, row_scale=dinv_g_b, bm=2000, bn=128, bk=128)
  agg_g = _sc_aggregate(y_g, idx3_g, zerosD)
  h_g = _conv_epilogue(agg_g[0], agg_g[1], y_g, dinv_g_b, bias_row(b_conv1))
  h_pool = _segment_max(h_g, oh_g)

  sel2 = jnp.concatenate([oh_x, rsel], axis=0)
  rs2 = jnp.concatenate([invcnt_x, jnp.ones((B, 128), f32)], axis=0)
  mr = _mm(sel2, x, row_scale=rs2, bm=32, bn=512, bk=128)
  cat1 = jnp.concatenate([mr[:B], mr[B:]], axis=1)
  new_x = _mm(cat1, W_lin1, bias=bias_row(b_lin1), act="prelu",
              alpha=alpha_arr, bm=16, bn=256, bk=128)
  new_x = _mm(new_x, W_lin2, bias=bias_row(b_lin2), act="prelu",
              alpha=alpha_arr, bm=16, bn=128, bk=256)

  Wc0p = jnp.pad(W_c0, ((0, DRAW_PAD - DRAW), (0, 0)))
  h = None
  pools = []
  y = _mm(x, Wc0p, row_scale=dinv_x_b, bm=2000, bn=128, bk=512)
  for Wc, bc in ((W_c0, b_c0), (W_c1, b_c1), (W_c2, b_c2)):
    if h is not None:
      y = _mm(h, Wc, row_scale=dinv_x_b, bm=1264, bn=128, bk=128)
    agg = _sc_aggregate(y, idx3_x, zerosD)
    h = _conv_epilogue(agg[0], agg[1], y, dinv_x_b, bias_row(bc))
    pools.append(_mm(oh_x, h, row_scale=invcnt_x, bm=16, bn=128, bk=128))

  cat = jnp.concatenate(pools + [new_x, h_pool], axis=1)
  W5p = jnp.pad(W_lin5, ((0, 0), (0, 128 - NCLS)))
  b5p = jnp.pad(b_lin5, (0, 128 - NCLS))
  out = _mm(cat, W5p, bias=bias_row(b5p), act="log_softmax",
            n_valid=NCLS, bm=16, bn=128, bk=128 * 5)
  return out[:, :NCLS]
